# Initial kernel scaffold; baseline (speedup 1.0000x reference)
import jax
import jax.numpy as jnp
from jax.experimental import pallas as pl


def kernel(x, params):
    raise NotImplementedError("write your pallas kernel here")


# speedup vs baseline: 2.9734x; 2.9734x over previous
import functools

import numpy as np

import jax
import jax.numpy as jnp
from jax.experimental import pallas as pl
from jax.experimental.pallas import tpu as pltpu

_B, _H, _W = 4, 128, 128
_PC, _PF = 8, 4
_ED, _HEADS, _DEPTH = 128, 4, 2
_HD = _ED // _HEADS
_Hc = _Wc = _H // _PC
_Hf = _Wf = _H // _PF
_Nc = _Hc * _Wc
_Nf = _Hf * _Wf
_N = _Nc + _Nf
_SCALE = _HD ** -0.5
_NLAYER_REFS = 11


def _unfold_patches(x, p):
    b, c, h, w = x.shape
    xr = x.reshape(b, c, h // p, p, w // p, p)
    xr = xr.transpose(0, 2, 4, 1, 3, 5)
    return xr.reshape(b, (h // p) * (w // p), c * p * p)


def _resize_mats():
    out = []
    for (oh, h) in ((_Hf, _Hc), (_Wf, _Wc)):
        ys = np.linspace(0.0, h - 1.0, oh).astype(np.float32)
        y0 = np.floor(ys).astype(np.int32)
        y1 = np.clip(y0 + 1, 0, h - 1)
        wy = (ys - y0).astype(np.float32)
        m = np.zeros((oh, h), np.float32)
        m[np.arange(oh), y0] += 1.0 - wy
        m[np.arange(oh), y1] += wy
        out.append(m)
    return out


_RY, _RX = _resize_mats()


def _ln(t, s, b):
    m = jnp.mean(t, axis=-1, keepdims=True)
    v = jnp.mean((t - m) ** 2, axis=-1, keepdims=True)
    return (t - m) * jax.lax.rsqrt(v + 1e-5) * s + b


def _fused_kernel(xs_ref, pc_ref, pf_ref, t0_ref, t1_ref,
                  cw_ref, cb_ref, fw_ref, fb_ref,
                  dcw_ref, dfw_ref, ry_ref, rx_ref, cp_ref,
                  *rest):
    layer_refs = rest[:_DEPTH * _NLAYER_REFS]
    out_ref = rest[_DEPTH * _NLAYER_REFS]
    pad_s, conv_s = rest[_DEPTH * _NLAYER_REFS + 1:]

    f32 = jnp.float32

    xv = xs_ref[0].astype(jnp.bfloat16).astype(f32)
    pad_s[...] = jnp.zeros((_H + 2, _W + 2), f32)
    pad_s[1:_H + 1, 1:_W + 1] = xv

    def sh(dy, dx):
        return pad_s[dy:dy + _H, dx:dx + _W]

    sx = (sh(0, 2) - sh(0, 0)) + 2.0 * (sh(1, 2) - sh(1, 0)) + (sh(2, 2) - sh(2, 0))
    sy = (sh(2, 0) - sh(0, 0)) + 2.0 * (sh(2, 1) - sh(0, 1)) + (sh(2, 2) - sh(0, 2))
    e = jnp.sqrt(sx * sx + sy * sy)

    pr = jax.lax.broadcasted_iota(jnp.int32, (_Hf, _H), 0)
    pc_i = jax.lax.broadcasted_iota(jnp.int32, (_Hf, _H), 1)
    pool = (pc_i // 4 == pr).astype(f32)
    hi = jax.lax.Precision.HIGHEST
    pooled = jnp.dot(jnp.dot(pool, e, preferred_element_type=f32, precision=hi),
                     pool.T, preferred_element_type=f32, precision=hi) * (1.0 / 16.0)
    mask2d = (pooled > jnp.mean(pooled)).astype(f32)

    ri = jax.lax.broadcasted_iota(jnp.int32, (_Nf, _Wf), 0)
    ci = jax.lax.broadcasted_iota(jnp.int32, (_Nf, _Wf), 1)
    rowsel = (ri // _Wf == ci).astype(f32)
    colsel = (ri % _Wf == ci).astype(f32)
    mvec = jnp.sum(jnp.dot(rowsel, mask2d, preferred_element_type=f32) * colsel,
                   axis=-1, keepdims=True)

    ct = jnp.dot(pc_ref[0], cw_ref[...], preferred_element_type=f32) + cb_ref[...] + t0_ref[...]
    ft = jnp.dot(pf_ref[0], fw_ref[...], preferred_element_type=f32) + fb_ref[...]
    gf = ft * mvec + t1_ref[...]
    tokens = jnp.concatenate([ct, gf], axis=0)

    for li in range(_DEPTH):
        (ln1s, ln1b, qkvw, projw, projb,
         ln2s, ln2b, w1, b1, w2, b2) = layer_refs[li * _NLAYER_REFS:(li + 1) * _NLAYER_REFS]
        h1 = _ln(tokens, ln1s[...], ln1b[...])
        qkv = jnp.dot(h1, qkvw[...], preferred_element_type=f32)
        outs = []
        for h in range(_HEADS):
            qh = qkv[:, h * _HD:(h + 1) * _HD]
            kh = qkv[:, _ED + h * _HD:_ED + (h + 1) * _HD]
            vh = qkv[:, 2 * _ED + h * _HD:2 * _ED + (h + 1) * _HD]
            att = jax.lax.dot_general(qh, kh, (((1,), (1,)), ((), ())),
                                      preferred_element_type=f32) * _SCALE
            att = att - jnp.max(att, axis=-1, keepdims=True)
            att = jnp.exp(att)
            att = att * (1.0 / jnp.sum(att, axis=-1, keepdims=True))
            outs.append(jnp.dot(att, vh, preferred_element_type=f32))
        o = jnp.concatenate(outs, axis=1)
        tokens = tokens + jnp.dot(o, projw[...], preferred_element_type=f32) + projb[...]
        h2 = _ln(tokens, ln2s[...], ln2b[...])
        a = jnp.dot(h2, w1[...], preferred_element_type=f32) + b1[...]
        g = 0.5 * a * (1.0 + jax.lax.erf(a * (0.5 ** 0.5)))
        tokens = tokens + jnp.dot(g, w2[...], preferred_element_type=f32) + b2[...]

    dec_cb = cp_ref[57]
    dec_fb = cp_ref[58]
    c_tok = tokens[:_Nc]
    f_tok = tokens[_Nc:]
    cvec = jnp.sum(c_tok * dcw_ref[...], axis=-1, keepdims=True) + dec_cb
    fvec = (jnp.sum(f_tok * dfw_ref[...], axis=-1, keepdims=True) + dec_fb) * mvec

    cri = jax.lax.broadcasted_iota(jnp.int32, (_Nc, _Wc), 0)
    cci = jax.lax.broadcasted_iota(jnp.int32, (_Nc, _Wc), 1)
    c_rowsel = (cri // _Wc == cci).astype(f32)
    c_colsel = (cri % _Wc == cci).astype(f32)
    c2d = jax.lax.dot_general(c_rowsel, cvec * c_colsel, (((0,), (0,)), ((), ())),
                              preferred_element_type=f32, precision=hi)
    f2d = jax.lax.dot_general(rowsel, fvec * colsel, (((0,), (0,)), ((), ())),
                              preferred_element_type=f32, precision=hi)

    cu = jnp.dot(ry_ref[...], c2d, preferred_element_type=f32, precision=hi)
    cu = jax.lax.dot_general(cu, rx_ref[...], (((1,), (1,)), ((), ())),
                             preferred_element_type=f32, precision=hi)

    conv_s[...] = jnp.zeros((2, _Hf + 2, _Wf + 2), f32)
    conv_s[0, 1:_Hf + 1, 1:_Wf + 1] = cu
    conv_s[1, 1:_Hf + 1, 1:_Wf + 1] = f2d

    def csh(i, dy, dx):
        return conv_s[i, dy:dy + _Hf, dx:dx + _Wf]

    ys = []
    for o in range(2):
        acc = jnp.full((_Hf, _Wf), cp_ref[36 + 2 * 9 + o])
        for i in range(2):
            for dy in range(3):
                for dx in range(3):
                    acc = acc + cp_ref[(o * 2 + i) * 9 + dy * 3 + dx] * csh(i, dy, dx)
        ys.append(jnp.maximum(acc, 0.0))
    conv_s[0, 1:_Hf + 1, 1:_Wf + 1] = ys[0]
    conv_s[1, 1:_Hf + 1, 1:_Wf + 1] = ys[1]

    acc = jnp.full((_Hf, _Wf), cp_ref[36 + 18 + 2])
    for i in range(2):
        for dy in range(3):
            for dx in range(3):
                acc = acc + cp_ref[36 + i * 9 + dy * 3 + dx] * csh(i, dy, dx)
    out_ref[0, 0] = acc


def kernel(x, params):
    f32 = jnp.float32
    xs = x[:, 0]
    pc = _unfold_patches(x, _PC)
    pf = _unfold_patches(x, _PF)

    row = lambda v: v.reshape(1, -1)
    t0 = row(params['type_embed'][0])
    t1 = row(params['type_embed'][1])

    cp = jnp.concatenate([
        params['fuse1_w'].reshape(-1), params['fuse2_w'].reshape(-1),
        params['fuse1_b'].reshape(-1), params['fuse2_b'].reshape(-1),
        params['dec_coarse_b'].reshape(-1), params['dec_fine_b'].reshape(-1),
    ]).astype(f32)

    layer_args = []
    for lp in params['layers']:
        layer_args += [row(lp['ln1_s']), row(lp['ln1_b']), lp['qkv_w'],
                       lp['proj_w'], row(lp['proj_b']),
                       row(lp['ln2_s']), row(lp['ln2_b']),
                       lp['mlp_w1'], row(lp['mlp_b1']),
                       lp['mlp_w2'], row(lp['mlp_b2'])]

    args = [xs, pc, pf, t0, t1,
            params['coarse_w'], row(params['coarse_b']),
            params['fine_w'], row(params['fine_b']),
            row(params['dec_coarse_w'][:, 0]), row(params['dec_fine_w'][:, 0]),
            jnp.asarray(_RY), jnp.asarray(_RX), cp] + layer_args

    def bspec(a):
        if a is xs:
            return pl.BlockSpec((1, _H, _W), lambda b: (b, 0, 0))
        if a is pc:
            return pl.BlockSpec((1, _Nc, _PC * _PC), lambda b: (b, 0, 0))
        if a is pf:
            return pl.BlockSpec((1, _Nf, _PF * _PF), lambda b: (b, 0, 0))
        if a is cp:
            return pl.BlockSpec(memory_space=pltpu.SMEM)
        nd = a.ndim
        return pl.BlockSpec(a.shape, lambda b: (0,) * nd)

    out = pl.pallas_call(
        _fused_kernel,
        grid=(_B,),
        in_specs=[bspec(a) for a in args],
        out_specs=pl.BlockSpec((1, 1, _Hf, _Wf), lambda b: (b, 0, 0, 0)),
        out_shape=jax.ShapeDtypeStruct((_B, 1, _Hf, _Wf), f32),
        scratch_shapes=[pltpu.VMEM((_H + 2, _W + 2), f32),
                        pltpu.VMEM((2, _Hf + 2, _Wf + 2), f32)],
        compiler_params=pltpu.CompilerParams(
            dimension_semantics=("parallel",)),
    )(*args)
    return out


# speedup vs baseline: 3.9762x; 1.3372x over previous
import functools

import numpy as np

import jax
import jax.numpy as jnp
from jax.experimental import pallas as pl
from jax.experimental.pallas import tpu as pltpu

_B, _H, _W = 4, 128, 128
_PC, _PF = 8, 4
_ED, _HEADS, _DEPTH = 128, 4, 2
_HD = _ED // _HEADS
_Hc = _Wc = _H // _PC
_Hf = _Wf = _H // _PF
_Nc = _Hc * _Wc
_Nf = _Hf * _Wf
_N = _Nc + _Nf
_SCALE = _HD ** -0.5
_NLAYER_REFS = 11


def _unfold_patches(x, p):
    b, c, h, w = x.shape
    xr = x.reshape(b, c, h // p, p, w // p, p)
    xr = xr.transpose(0, 2, 4, 1, 3, 5)
    return xr.reshape(b, (h // p) * (w // p), c * p * p)


def _resize_mats():
    out = []
    for (oh, h) in ((_Hf, _Hc), (_Wf, _Wc)):
        ys = np.linspace(0.0, h - 1.0, oh).astype(np.float32)
        y0 = np.floor(ys).astype(np.int32)
        y1 = np.clip(y0 + 1, 0, h - 1)
        wy = (ys - y0).astype(np.float32)
        m = np.zeros((oh, h), np.float32)
        m[np.arange(oh), y0] += 1.0 - wy
        m[np.arange(oh), y1] += wy
        out.append(m)
    return out


_RY, _RX = _resize_mats()


def _ln(t, s, b):
    m = jnp.mean(t, axis=-1, keepdims=True)
    v = jnp.mean((t - m) ** 2, axis=-1, keepdims=True)
    return (t - m) * jax.lax.rsqrt(v + 1e-5) * s + b


def _fused_kernel(xs_ref, pc_ref, pf_ref, t0_ref, t1_ref,
                  cw_ref, cb_ref, fw_ref, fb_ref,
                  dcw_ref, dfw_ref, ry_ref, rx_ref, cp_ref,
                  *rest):
    layer_refs = rest[:_DEPTH * _NLAYER_REFS]
    out_ref = rest[_DEPTH * _NLAYER_REFS]
    pad_s, conv_s = rest[_DEPTH * _NLAYER_REFS + 1:]

    f32 = jnp.float32

    xv = xs_ref[0].astype(jnp.bfloat16).astype(f32)
    pad_s[...] = jnp.zeros((_H + 2, _W + 2), f32)
    pad_s[1:_H + 1, 1:_W + 1] = xv

    def sh(dy, dx):
        return pad_s[dy:dy + _H, dx:dx + _W]

    sx = (sh(0, 2) - sh(0, 0)) + 2.0 * (sh(1, 2) - sh(1, 0)) + (sh(2, 2) - sh(2, 0))
    sy = (sh(2, 0) - sh(0, 0)) + 2.0 * (sh(2, 1) - sh(0, 1)) + (sh(2, 2) - sh(0, 2))
    e = jnp.sqrt(sx * sx + sy * sy)

    pr = jax.lax.broadcasted_iota(jnp.int32, (_Hf, _H), 0)
    pc_i = jax.lax.broadcasted_iota(jnp.int32, (_Hf, _H), 1)
    pool = (pc_i // 4 == pr).astype(f32)
    hi = jax.lax.Precision.HIGHEST
    pooled = jnp.dot(jnp.dot(pool, e, preferred_element_type=f32, precision=hi),
                     pool.T, preferred_element_type=f32, precision=hi) * (1.0 / 16.0)
    mask2d = (pooled > jnp.mean(pooled)).astype(f32)

    ri = jax.lax.broadcasted_iota(jnp.int32, (_Nf, _Wf), 0)
    ci = jax.lax.broadcasted_iota(jnp.int32, (_Nf, _Wf), 1)
    rowsel = (ri // _Wf == ci).astype(f32)
    colsel = (ri % _Wf == ci).astype(f32)
    mvec = jnp.sum(jnp.dot(rowsel, mask2d, preferred_element_type=f32) * colsel,
                   axis=-1, keepdims=True)

    ct = jnp.dot(pc_ref[0], cw_ref[...], preferred_element_type=f32) + cb_ref[...] + t0_ref[...]
    ft = jnp.dot(pf_ref[0], fw_ref[...], preferred_element_type=f32) + fb_ref[...]
    gf = ft * mvec + t1_ref[...]
    tokens = jnp.concatenate([ct, gf], axis=0)

    for li in range(_DEPTH):
        (ln1s, ln1b, qkvw, projw, projb,
         ln2s, ln2b, w1, b1, w2, b2) = layer_refs[li * _NLAYER_REFS:(li + 1) * _NLAYER_REFS]
        h1 = _ln(tokens, ln1s[...], ln1b[...])
        qkv = jnp.dot(h1, qkvw[...], preferred_element_type=f32)
        outs = []
        for h in range(_HEADS):
            qh = qkv[:, h * _HD:(h + 1) * _HD] * _SCALE
            kh = qkv[:, _ED + h * _HD:_ED + (h + 1) * _HD]
            vh = qkv[:, 2 * _ED + h * _HD:2 * _ED + (h + 1) * _HD]
            att = jax.lax.dot_general(qh, kh, (((1,), (1,)), ((), ())),
                                      preferred_element_type=f32)
            att = jnp.exp(jnp.clip(att, -80.0, 80.0))
            vaug = jnp.concatenate([vh, jnp.ones((_N, 1), f32)], axis=1)
            oa = jnp.dot(att, vaug, preferred_element_type=f32)
            outs.append(oa[:, :_HD] * (1.0 / oa[:, _HD:_HD + 1]))
        o = jnp.concatenate(outs, axis=1)
        tokens = tokens + jnp.dot(o, projw[...], preferred_element_type=f32) + projb[...]
        h2 = _ln(tokens, ln2s[...], ln2b[...])
        a = jnp.dot(h2, w1[...], preferred_element_type=f32) + b1[...]
        g = 0.5 * a * (1.0 + jax.lax.erf(a * (0.5 ** 0.5)))
        tokens = tokens + jnp.dot(g, w2[...], preferred_element_type=f32) + b2[...]

    dec_cb = cp_ref[57]
    dec_fb = cp_ref[58]
    c_tok = tokens[:_Nc]
    f_tok = tokens[_Nc:]
    cvec = jnp.sum(c_tok * dcw_ref[...], axis=-1, keepdims=True) + dec_cb
    fvec = (jnp.sum(f_tok * dfw_ref[...], axis=-1, keepdims=True) + dec_fb) * mvec

    cri = jax.lax.broadcasted_iota(jnp.int32, (_Nc, _Wc), 0)
    cci = jax.lax.broadcasted_iota(jnp.int32, (_Nc, _Wc), 1)
    c_rowsel = (cri // _Wc == cci).astype(f32)
    c_colsel = (cri % _Wc == cci).astype(f32)
    c2d = jax.lax.dot_general(c_rowsel, cvec * c_colsel, (((0,), (0,)), ((), ())),
                              preferred_element_type=f32, precision=hi)
    f2d = jax.lax.dot_general(rowsel, fvec * colsel, (((0,), (0,)), ((), ())),
                              preferred_element_type=f32, precision=hi)

    cu = jnp.dot(ry_ref[...], c2d, preferred_element_type=f32, precision=hi)
    cu = jax.lax.dot_general(cu, rx_ref[...], (((1,), (1,)), ((), ())),
                             preferred_element_type=f32, precision=hi)

    conv_s[...] = jnp.zeros((2, _Hf + 2, _Wf + 2), f32)
    conv_s[0, 1:_Hf + 1, 1:_Wf + 1] = cu
    conv_s[1, 1:_Hf + 1, 1:_Wf + 1] = f2d

    def csh(i, dy, dx):
        return conv_s[i, dy:dy + _Hf, dx:dx + _Wf]

    ys = []
    for o in range(2):
        acc = jnp.full((_Hf, _Wf), cp_ref[36 + 2 * 9 + o])
        for i in range(2):
            for dy in range(3):
                for dx in range(3):
                    acc = acc + cp_ref[(o * 2 + i) * 9 + dy * 3 + dx] * csh(i, dy, dx)
        ys.append(jnp.maximum(acc, 0.0))
    conv_s[0, 1:_Hf + 1, 1:_Wf + 1] = ys[0]
    conv_s[1, 1:_Hf + 1, 1:_Wf + 1] = ys[1]

    acc = jnp.full((_Hf, _Wf), cp_ref[36 + 18 + 2])
    for i in range(2):
        for dy in range(3):
            for dx in range(3):
                acc = acc + cp_ref[36 + i * 9 + dy * 3 + dx] * csh(i, dy, dx)
    out_ref[0, 0] = acc


def kernel(x, params):
    f32 = jnp.float32
    xs = x[:, 0]
    pc = _unfold_patches(x, _PC)
    pf = _unfold_patches(x, _PF)

    row = lambda v: v.reshape(1, -1)
    t0 = row(params['type_embed'][0])
    t1 = row(params['type_embed'][1])

    cp = jnp.concatenate([
        params['fuse1_w'].reshape(-1), params['fuse2_w'].reshape(-1),
        params['fuse1_b'].reshape(-1), params['fuse2_b'].reshape(-1),
        params['dec_coarse_b'].reshape(-1), params['dec_fine_b'].reshape(-1),
    ]).astype(f32)

    layer_args = []
    for lp in params['layers']:
        layer_args += [row(lp['ln1_s']), row(lp['ln1_b']), lp['qkv_w'],
                       lp['proj_w'], row(lp['proj_b']),
                       row(lp['ln2_s']), row(lp['ln2_b']),
                       lp['mlp_w1'], row(lp['mlp_b1']),
                       lp['mlp_w2'], row(lp['mlp_b2'])]

    args = [xs, pc, pf, t0, t1,
            params['coarse_w'], row(params['coarse_b']),
            params['fine_w'], row(params['fine_b']),
            row(params['dec_coarse_w'][:, 0]), row(params['dec_fine_w'][:, 0]),
            jnp.asarray(_RY), jnp.asarray(_RX), cp] + layer_args

    def bspec(a):
        if a is xs:
            return pl.BlockSpec((1, _H, _W), lambda b: (b, 0, 0))
        if a is pc:
            return pl.BlockSpec((1, _Nc, _PC * _PC), lambda b: (b, 0, 0))
        if a is pf:
            return pl.BlockSpec((1, _Nf, _PF * _PF), lambda b: (b, 0, 0))
        if a is cp:
            return pl.BlockSpec(memory_space=pltpu.SMEM)
        nd = a.ndim
        return pl.BlockSpec(a.shape, lambda b: (0,) * nd)

    out = pl.pallas_call(
        _fused_kernel,
        grid=(_B,),
        in_specs=[bspec(a) for a in args],
        out_specs=pl.BlockSpec((1, 1, _Hf, _Wf), lambda b: (b, 0, 0, 0)),
        out_shape=jax.ShapeDtypeStruct((_B, 1, _Hf, _Wf), f32),
        scratch_shapes=[pltpu.VMEM((_H + 2, _W + 2), f32),
                        pltpu.VMEM((2, _Hf + 2, _Wf + 2), f32)],
        compiler_params=pltpu.CompilerParams(
            dimension_semantics=("parallel",)),
    )(*args)
    return out
